# baseline (speedup 1.0000x reference)
import jax
import jax.numpy as jnp
from jax import lax
from jax.experimental import pallas as pl
from jax.experimental.pallas import tpu as pltpu
from jax.experimental.pallas import tpu_sc as plsc

N_NODES = 100000
D = 128
NC = 2
NS = 16
NW = NC * NS
L = 16

ROWS_W = 3120
MAIN = NW * ROWS_W
CH = 80
NCHUNK = ROWS_W // CH
NBUF = 3
TAIL = N_NODES - MAIN
TAIL_W = TAIL // 8


def _splat_lane(tv, rl):
    dnums = lax.GatherDimensionNumbers(
        offset_dims=(), collapsed_slice_dims=(0,), start_index_map=(0,))
    return lax.gather(
        tv, jnp.full((L, 1), rl, jnp.int32), dnums, (1,),
        mode=lax.GatherScatterMode.PROMISE_IN_BOUNDS)


def _add_row(xb, tbl_v, t, rowflat):
    for j in range(D // L):
        col = jnp.arange(j * L, (j + 1) * L, dtype=jnp.int32)
        v = plsc.load_gather(tbl_v, [t + col])
        plsc.addupdate(xb.at[pl.ds(rowflat + j * L, L)], v)


def _body(x_hbm, idx_hbm, tbl_hbm, out_hbm, idx_v, tbl_v,
          xb0, xb1, xb2, tidx_v, txb,
          sx0, sx1, sx2, so0, so1, so2, st, stx):
    xb = [xb0, xb1, xb2]
    sx = [sx0, sx1, sx2]
    so = [so0, so1, so2]

    wid = lax.axis_index("s") * NC + lax.axis_index("c")
    base = wid * ROWS_W
    t0 = MAIN + wid * 8

    def start_in(k):
        s = k % NBUF
        return pltpu.async_copy(
            x_hbm.at[pl.ds((base + k * CH) * D, CH * D)], xb[s], sx[s])

    in_desc = [None] * NCHUNK
    out_desc = [None] * NCHUNK
    in_desc[0] = start_in(0)
    in_desc[1] = start_in(1)
    in_desc[2] = start_in(2)
    tail_on = wid < TAIL_W

    @pl.when(tail_on)
    def _tail_prefetch():
        pltpu.async_copy(idx_hbm.at[pl.ds(t0 - 8, L)], tidx_v, st)
        pltpu.async_copy(x_hbm.at[pl.ds(t0 * D, 8 * D)], txb, stx)

    pltpu.sync_copy(tbl_hbm, tbl_v)
    pltpu.sync_copy(idx_hbm.at[pl.ds(base, ROWS_W)], idx_v.at[pl.ds(0, ROWS_W)])

    for k in range(NCHUNK):
        s = k % NBUF
        in_desc[k].wait()

        @plsc.parallel_loop(0, CH, unroll=2)
        def chunk_rows(r, k=k, s=s):
            tv = idx_v[pl.ds(k * CH + r, L)]
            t = _splat_lane(tv, 0) * D
            _add_row(xb[s], tbl_v, t, r * D)

        out_desc[k] = pltpu.async_copy(
            xb[s], out_hbm.at[pl.ds((base + k * CH) * D, CH * D)], so[s])
        if k >= 1 and k + 2 < NCHUNK:
            out_desc[k - 1].wait()
            in_desc[k + 2] = start_in(k + 2)
    @pl.when(tail_on)
    def _tail():
        pltpu.make_async_copy(idx_hbm.at[pl.ds(t0 - 8, L)], tidx_v, st).wait()
        pltpu.make_async_copy(x_hbm.at[pl.ds(t0 * D, 8 * D)], txb, stx).wait()
        tv = tidx_v[...]
        for rl in range(8):
            t = _splat_lane(tv, 8 + rl) * D
            _add_row(txb, tbl_v, t, rl * D)
        pltpu.sync_copy(txb, out_hbm.at[pl.ds(t0 * D, 8 * D)])

    for k in range(max(0, NCHUNK - 3), NCHUNK):
        out_desc[k].wait()


@jax.jit
def _sc_add_embed(x, idx, tbl):
    mesh = plsc.VectorSubcoreMesh(
        core_axis_name="c", subcore_axis_name="s", num_cores=NC, num_subcores=NS
    )
    return pl.kernel(
        _body,
        out_type=jax.ShapeDtypeStruct((N_NODES * D,), jnp.float32),
        mesh=mesh,
        compiler_params=pltpu.CompilerParams(needs_layout_passes=False),
        scratch_types=[
            pltpu.VMEM((ROWS_W + L,), jnp.int32),
            pltpu.VMEM((64 * D,), jnp.float32),
            pltpu.VMEM((CH * D,), jnp.float32),
            pltpu.VMEM((CH * D,), jnp.float32),
            pltpu.VMEM((CH * D,), jnp.float32),
            pltpu.VMEM((L,), jnp.int32),
            pltpu.VMEM((8 * D,), jnp.float32),
            pltpu.SemaphoreType.DMA,
            pltpu.SemaphoreType.DMA,
            pltpu.SemaphoreType.DMA,
            pltpu.SemaphoreType.DMA,
            pltpu.SemaphoreType.DMA,
            pltpu.SemaphoreType.DMA,
            pltpu.SemaphoreType.DMA,
            pltpu.SemaphoreType.DMA,
        ],
    )(x, idx, tbl)


def kernel(x, node_types, type_table):
    out = _sc_add_embed(
        x.reshape(-1), node_types.astype(jnp.int32), type_table.reshape(-1))
    return out.reshape(N_NODES, D)


# speedup vs baseline: 1.2190x; 1.2190x over previous
import jax
import jax.numpy as jnp
from jax import lax
from jax.experimental import pallas as pl
from jax.experimental.pallas import tpu as pltpu
from jax.experimental.pallas import tpu_sc as plsc

N_NODES = 100000
D = 128
NC = 2
NS = 16
NW = NC * NS
L = 16

ROWS_W = 3120
MAIN = NW * ROWS_W
CH = 240
NCHUNK = ROWS_W // CH
NBUF = 3
TAIL = N_NODES - MAIN
TAIL_W = TAIL // 8


def _splat_lane(tv, rl):
    dnums = lax.GatherDimensionNumbers(
        offset_dims=(), collapsed_slice_dims=(0,), start_index_map=(0,))
    return lax.gather(
        tv, jnp.full((L, 1), rl, jnp.int32), dnums, (1,),
        mode=lax.GatherScatterMode.PROMISE_IN_BOUNDS)


def _add_row(xb, tbl_v, t, rowflat):
    for j in range(D // L):
        col = jnp.arange(j * L, (j + 1) * L, dtype=jnp.int32)
        v = plsc.load_gather(tbl_v, [t + col])
        plsc.addupdate(xb.at[pl.ds(rowflat + j * L, L)], v)


def _body(x_hbm, idx_hbm, tbl_hbm, out_hbm, idx_v, tbl_v,
          xb0, xb1, xb2, tidx_v, txb,
          sx0, sx1, sx2, so0, so1, so2, st, stx):
    xb = [xb0, xb1, xb2]
    sx = [sx0, sx1, sx2]
    so = [so0, so1, so2]

    wid = lax.axis_index("s") * NC + lax.axis_index("c")
    base = wid * ROWS_W
    t0 = MAIN + wid * 8

    def start_in(k):
        s = k % NBUF
        return pltpu.async_copy(
            x_hbm.at[pl.ds((base + k * CH) * D, CH * D)], xb[s], sx[s])

    in_desc = [None] * NCHUNK
    out_desc = [None] * NCHUNK
    in_desc[0] = start_in(0)
    in_desc[1] = start_in(1)
    in_desc[2] = start_in(2)
    tail_on = wid < TAIL_W

    @pl.when(tail_on)
    def _tail_prefetch():
        pltpu.async_copy(idx_hbm.at[pl.ds(t0 - 8, L)], tidx_v, st)
        pltpu.async_copy(x_hbm.at[pl.ds(t0 * D, 8 * D)], txb, stx)

    pltpu.sync_copy(tbl_hbm, tbl_v)
    pltpu.sync_copy(idx_hbm.at[pl.ds(base, ROWS_W)], idx_v.at[pl.ds(0, ROWS_W)])

    for k in range(NCHUNK):
        s = k % NBUF
        in_desc[k].wait()


        @plsc.parallel_loop(0, CH, unroll=3)
        def chunk_rows(r, k=k, s=s):
            tv = idx_v[pl.ds(k * CH + r, L)]
            t = _splat_lane(tv, 0) * D
            _add_row(xb[s], tbl_v, t, r * D)

        out_desc[k] = pltpu.async_copy(
            xb[s], out_hbm.at[pl.ds((base + k * CH) * D, CH * D)], so[s])
        if k >= 1 and k + 2 < NCHUNK:
            out_desc[k - 1].wait()
            in_desc[k + 2] = start_in(k + 2)
    @pl.when(tail_on)
    def _tail():
        pltpu.make_async_copy(idx_hbm.at[pl.ds(t0 - 8, L)], tidx_v, st).wait()
        pltpu.make_async_copy(x_hbm.at[pl.ds(t0 * D, 8 * D)], txb, stx).wait()
        tD = tidx_v[...] * D
        for rl in range(8):
            t = _splat_lane(tD, 8 + rl)
            _add_row(txb, tbl_v, t, rl * D)
        pltpu.sync_copy(txb, out_hbm.at[pl.ds(t0 * D, 8 * D)])

    for k in range(max(0, NCHUNK - 3), NCHUNK):
        out_desc[k].wait()


@jax.jit
def _sc_add_embed(x, idx, tbl):
    mesh = plsc.VectorSubcoreMesh(
        core_axis_name="c", subcore_axis_name="s", num_cores=NC, num_subcores=NS
    )
    return pl.kernel(
        _body,
        out_type=jax.ShapeDtypeStruct((N_NODES * D,), jnp.float32),
        mesh=mesh,
        compiler_params=pltpu.CompilerParams(needs_layout_passes=False),
        scratch_types=[
            pltpu.VMEM((ROWS_W + L,), jnp.int32),
            pltpu.VMEM((64 * D,), jnp.float32),
            pltpu.VMEM((CH * D,), jnp.float32),
            pltpu.VMEM((CH * D,), jnp.float32),
            pltpu.VMEM((CH * D,), jnp.float32),
            pltpu.VMEM((L,), jnp.int32),
            pltpu.VMEM((8 * D,), jnp.float32),
            pltpu.SemaphoreType.DMA,
            pltpu.SemaphoreType.DMA,
            pltpu.SemaphoreType.DMA,
            pltpu.SemaphoreType.DMA,
            pltpu.SemaphoreType.DMA,
            pltpu.SemaphoreType.DMA,
            pltpu.SemaphoreType.DMA,
            pltpu.SemaphoreType.DMA,
        ],
    )(x, idx, tbl)


def kernel(x, node_types, type_table):
    out = _sc_add_embed(
        x.reshape(-1), node_types.astype(jnp.int32), type_table.reshape(-1))
    return out.reshape(N_NODES, D)
